# baseline (speedup 1.0000x reference)
import functools

import jax
import jax.numpy as jnp
import numpy as np
from jax.experimental import pallas as pl

_BNS = 1.0 / np.sqrt(1.0 + 1e-5)

_SA1 = [[32, 32, 64], [64, 64, 128], [64, 96, 128]]
_SA2 = [[128, 128, 256], [128, 196, 256]]


def _fold(p, name):
    w = p[name + '_w']
    b = p[name + '_b']
    if name + '_g' in p:
        s = _BNS * p[name + '_g']
        w = w * s[:, None]
        b = b * s + p[name + '_be']
    return w.T, b[None, :]


def _cumsum_lanes(x, n):
    sh = 1
    while sh < n:
        z = jnp.zeros(x.shape[:-1] + (sh,), x.dtype)
        x = x + jnp.concatenate([z, x[..., :n - sh]], axis=-1)
        sh *= 2
    return x


def _tile_rows(x, k):
    while k > 1:
        x = jnp.concatenate([x, x], axis=0)
        k //= 2
    return x


def _fps_body(x_ref, o_ref, *, npoint, n):
    bsz = x_ref.shape[0]
    x = x_ref[...]
    iota = jax.lax.broadcasted_iota(jnp.int32, (bsz, n), 1)

    def step(i, carry):
        dist, far = carry
        onehot = (iota == far).astype(jnp.float32)
        cent = jnp.sum(x * onehot[:, None, :], axis=2)
        o_ref[:, pl.ds(i, 1), :] = cent[:, None, :]
        d = jnp.sum((x - cent[:, :, None]) ** 2, axis=1)
        dist = jnp.minimum(dist, d)
        m = jnp.max(dist, axis=1, keepdims=True)
        far = jnp.min(jnp.where(dist == m, iota, n), axis=1, keepdims=True)
        return dist, far

    jax.lax.fori_loop(
        0, npoint, step,
        (jnp.full((bsz, n), 1e10, jnp.float32),
         jnp.zeros((bsz, 1), jnp.int32)))


def _fps(xyz_c, npoint):
    bsz, _, n = xyz_c.shape
    return pl.pallas_call(
        functools.partial(_fps_body, npoint=npoint, n=n),
        out_shape=jax.ShapeDtypeStruct((bsz, npoint, 3), jnp.float32),
    )(xyz_c)


def _sa_body(nx_ref, xtt_ref, *refs, r2, k, ts, n, nlayers, has_pt):
    pos = 0
    pt_ref = None
    if has_pt:
        pt_ref = refs[pos]
        pos += 1
    w1x_ref = refs[pos]
    pos += 1
    wbs = refs[pos:pos + 2 * nlayers]
    o_ref = refs[-1]

    nx = nx_ref[0]
    xtt = xtt_ref[0]
    xn2 = jnp.sum(xtt * xtt, axis=0, keepdims=True)
    d = (jnp.sum(nx * nx, axis=1, keepdims=True) + xn2
         - 2.0 * jnp.dot(nx, xtt, preferred_element_type=jnp.float32))
    ci = _cumsum_lanes((d <= r2).astype(jnp.int32), n)
    cm = jnp.where(d <= r2, ci, 0)

    rows = ts * k
    rio = jax.lax.broadcasted_iota(jnp.int32, (rows, 1), 0)
    ki = rio // ts + 1
    cmr = _tile_rows(cm, k)
    oh = (cmr == ki)
    empty = _tile_rows(ci[:, n - 1:n] == 0, k)
    lane = jax.lax.broadcasted_iota(jnp.int32, (rows, n), 1)
    oh = (oh | (empty & (lane == n - 1))).astype(jnp.float32)

    w1x = w1x_ref[...]
    rxt = jnp.dot(oh, xtt.T, preferred_element_type=jnp.float32,
                  precision=jax.lax.Precision.HIGHEST)
    sub = _tile_rows(
        jnp.dot(nx, w1x, preferred_element_type=jnp.float32), k)
    w1, b1 = wbs[0][...], wbs[1][...]
    if has_pt:
        rpt = jnp.dot(oh, pt_ref[0], preferred_element_type=jnp.float32,
                      precision=jax.lax.Precision.HIGHEST)
        h = (jnp.dot(rpt, w1, preferred_element_type=jnp.float32)
             + jnp.dot(rxt, w1x, preferred_element_type=jnp.float32))
    else:
        h = jnp.dot(rxt, w1, preferred_element_type=jnp.float32)
    h = jnp.maximum(h - sub + b1, 0.0)
    for j in range(1, nlayers):
        w, b = wbs[2 * j][...], wbs[2 * j + 1][...]
        h = jnp.maximum(
            jnp.dot(h, w, preferred_element_type=jnp.float32) + b, 0.0)

    h = h + (jnp.sum(oh, axis=1, keepdims=True) - 1.0) * 1e30
    while rows > ts:
        rows //= 2
        h = jnp.maximum(h[:rows], h[rows:])
    o_ref[0] = h


def _sa_branch(nx, xtc, pt, radius, k, ts, wbs):
    bsz, s, _ = nx.shape
    n = xtc.shape[2]
    cout = wbs[-1][0].shape[1]
    nlayers = len(wbs)
    w1 = wbs[0][0]
    w1x = w1[-3:]
    in_specs = [
        pl.BlockSpec((1, ts, 3), lambda bi, ti: (bi, ti, 0)),
        pl.BlockSpec((1, 3, n), lambda bi, ti: (bi, 0, 0)),
    ]
    args = [nx, xtc]
    if pt is not None:
        in_specs.append(
            pl.BlockSpec((1, n, pt.shape[2]), lambda bi, ti: (bi, 0, 0)))
        args.append(pt)
        w1p = w1[:-3]
    else:
        w1p = w1[:3] + w1x
    in_specs.append(pl.BlockSpec(w1x.shape, lambda bi, ti: (0, 0)))
    args.append(w1x)
    wlist = [(w1p, wbs[0][1])] + wbs[1:]
    for wt, b in wlist:
        in_specs.append(pl.BlockSpec(wt.shape, lambda bi, ti: (0, 0)))
        in_specs.append(pl.BlockSpec(b.shape, lambda bi, ti: (0, 0)))
        args += [wt, b]
    return pl.pallas_call(
        functools.partial(
            _sa_body, r2=float(np.float32(radius) ** 2), k=k, ts=ts, n=n,
            nlayers=nlayers, has_pt=pt is not None),
        grid=(bsz, s // ts),
        in_specs=in_specs,
        out_specs=pl.BlockSpec((1, ts, cout), lambda bi, ti: (bi, ti, 0)),
        out_shape=jax.ShapeDtypeStruct((bsz, s, cout), jnp.float32),
    )(*args)


def _sa3fp3_body(x_ref, *refs):
    o_ref = refs[-1]
    x = x_ref[0]
    h = x
    for j in range(3):
        w = refs[2 * j][...]
        b = refs[2 * j + 1][...]
        h = jnp.maximum(
            jnp.dot(h, w, preferred_element_type=jnp.float32) + b, 0.0)
    g = jnp.max(h, axis=0, keepdims=True)
    f = jnp.concatenate(
        [x[:, 3:], jnp.broadcast_to(g, (x.shape[0], g.shape[1]))], axis=1)
    for j in range(3, 5):
        w = refs[2 * j][...]
        b = refs[2 * j + 1][...]
        f = jnp.maximum(
            jnp.dot(f, w, preferred_element_type=jnp.float32) + b, 0.0)
    o_ref[0] = f


def _sa3_fp3(l2cat, wbs):
    bsz, s, c = l2cat.shape
    cout = wbs[-1][0].shape[1]
    in_specs = [pl.BlockSpec((1, s, c), lambda bi: (bi, 0, 0))]
    args = [l2cat]
    for wt, b in wbs:
        in_specs.append(pl.BlockSpec(wt.shape, lambda bi: (0, 0)))
        in_specs.append(pl.BlockSpec(b.shape, lambda bi: (0, 0)))
        args += [wt, b]
    return pl.pallas_call(
        _sa3fp3_body,
        grid=(bsz,),
        in_specs=in_specs,
        out_specs=pl.BlockSpec((1, s, cout), lambda bi: (bi, 0, 0)),
        out_shape=jax.ShapeDtypeStruct((bsz, s, cout), jnp.float32),
    )(*args)


def _fp_body(x1_ref, p1_ref, x2_ref, p2_ref, *refs, nrelu, nlin, s):
    o_ref = refs[-1]
    x1 = x1_ref[0]
    x2 = x2_ref[0]
    tn = x1.shape[0]
    d = (jnp.sum(x1 * x1, axis=1, keepdims=True)
         + jnp.sum(x2 * x2, axis=1, keepdims=True).T
         - 2.0 * jnp.dot(x1, x2.T, preferred_element_type=jnp.float32))
    iota = jax.lax.broadcasted_iota(jnp.int32, (tn, s), 1)
    oh = jnp.zeros((tn, s), jnp.float32)
    recips = []
    onehots = []
    for _ in range(3):
        m = jnp.min(d, axis=1, keepdims=True)
        idx = jnp.min(jnp.where(d == m, iota, s), axis=1, keepdims=True)
        hit = (iota == idx)
        recips.append(1.0 / (m + 1e-8))
        onehots.append(hit)
        d = jnp.where(hit, jnp.inf, d)
    wsum = recips[0] + recips[1] + recips[2]
    for r, hit in zip(recips, onehots):
        oh = oh + jnp.where(hit, jnp.broadcast_to(r / wsum, hit.shape), 0.0)
    interp = jnp.dot(oh, p2_ref[0], preferred_element_type=jnp.float32,
                     precision=jax.lax.Precision.HIGHEST)
    h = jnp.concatenate([p1_ref[0], interp], axis=1)
    for j in range(nrelu + nlin):
        w = refs[2 * j][...]
        b = refs[2 * j + 1][...]
        h = jnp.dot(h, w, preferred_element_type=jnp.float32) + b
        if j < nrelu:
            h = jnp.maximum(h, 0.0)
    o_ref[0] = h


def _fp(x1, p1, x2, p2, wbs, nlin, tn):
    bsz, n, _ = x1.shape
    s = x2.shape[1]
    cout = wbs[-1][0].shape[1]
    in_specs = [
        pl.BlockSpec((1, tn, 3), lambda bi, ti: (bi, ti, 0)),
        pl.BlockSpec((1, tn, p1.shape[2]), lambda bi, ti: (bi, ti, 0)),
        pl.BlockSpec((1, s, 3), lambda bi, ti: (bi, 0, 0)),
        pl.BlockSpec((1, s, p2.shape[2]), lambda bi, ti: (bi, 0, 0)),
    ]
    args = [x1, p1, x2, p2]
    for wt, b in wbs:
        in_specs.append(pl.BlockSpec(wt.shape, lambda bi, ti: (0, 0)))
        in_specs.append(pl.BlockSpec(b.shape, lambda bi, ti: (0, 0)))
        args += [wt, b]
    return pl.pallas_call(
        functools.partial(_fp_body, nrelu=len(wbs) - nlin, nlin=nlin, s=s),
        grid=(bsz, n // tn),
        in_specs=in_specs,
        out_specs=pl.BlockSpec((1, tn, cout), lambda bi, ti: (bi, ti, 0)),
        out_shape=jax.ShapeDtypeStruct((bsz, n, cout), jnp.float32),
    )(*args)


def kernel(xyz, params):
    p = params
    xt = jnp.transpose(xyz, (0, 2, 1))

    nx1 = _fps(xyz, 512)
    outs = []
    for bi, (radius, k, ts) in enumerate(
            [(0.1, 32, 32), (0.2, 64, 16), (0.4, 128, 8)]):
        wbs = [_fold(p, 'sa1_%d_%d' % (bi, j)) for j in range(len(_SA1[bi]))]
        outs.append(_sa_branch(nx1, xyz, None, radius, k, ts, wbs))
    l1p = jnp.concatenate(outs, axis=-1)

    nx1c = jnp.transpose(nx1, (0, 2, 1))
    nx2 = _fps(nx1c, 128)
    outs = []
    for bi, (radius, k, ts) in enumerate([(0.4, 64, 16), (0.8, 128, 8)]):
        wbs = [_fold(p, 'sa2_%d_%d' % (bi, j)) for j in range(len(_SA2[bi]))]
        outs.append(_sa_branch(nx2, nx1c, l1p, radius, k, ts, wbs))
    l2p = jnp.concatenate(outs, axis=-1)

    l2cat = jnp.concatenate([nx2, l2p], axis=-1)
    wbs = ([_fold(p, 'sa3_%d' % j) for j in range(3)]
           + [_fold(p, 'fp3_%d' % j) for j in range(2)])
    l2p = _sa3_fp3(l2cat, wbs)

    wbs = [_fold(p, 'fp2_%d' % j) for j in range(2)]
    l1p = _fp(nx1, l1p, nx2, l2p, wbs, nlin=0, tn=512)

    wbs = ([_fold(p, 'fp1_%d' % j) for j in range(2)]
           + [_fold(p, 'conv1'), _fold(p, 'conv2')])
    return _fp(xt, xt, nx1, l1p, wbs, nlin=1, tn=512)


# speedup vs baseline: 1.0851x; 1.0851x over previous
import functools

import jax
import jax.numpy as jnp
import numpy as np
from jax.experimental import pallas as pl

_BNS = 1.0 / np.sqrt(1.0 + 1e-5)

_SA1 = [[32, 32, 64], [64, 64, 128], [64, 96, 128]]
_SA2 = [[128, 128, 256], [128, 196, 256]]


def _fold(p, name):
    w = p[name + '_w']
    b = p[name + '_b']
    if name + '_g' in p:
        s = _BNS * p[name + '_g']
        w = w * s[:, None]
        b = b * s + p[name + '_be']
    return w.T, b[None, :]


def _cumsum_lanes(x, n):
    sh = 1
    while sh < n:
        z = jnp.zeros(x.shape[:-1] + (sh,), x.dtype)
        x = x + jnp.concatenate([z, x[..., :n - sh]], axis=-1)
        sh *= 2
    return x


def _tile_rows(x, k):
    while k > 1:
        x = jnp.concatenate([x, x], axis=0)
        k //= 2
    return x


def _fps_body(x_ref, o_ref, *, npoint, n):
    bsz = x_ref.shape[0]
    x = x_ref[...]
    iota = jax.lax.broadcasted_iota(jnp.int32, (bsz, n), 1)

    def step(i, carry):
        dist, far = carry
        onehot = (iota == far).astype(jnp.float32)
        cent = jnp.sum(x * onehot[:, None, :], axis=2)
        o_ref[:, pl.ds(i, 1), :] = cent[:, None, :]
        d = jnp.sum((x - cent[:, :, None]) ** 2, axis=1)
        dist = jnp.minimum(dist, d)
        m = jnp.max(dist, axis=1, keepdims=True)
        far = jnp.min(jnp.where(dist == m, iota, n), axis=1, keepdims=True)
        return dist, far

    jax.lax.fori_loop(
        0, npoint, step,
        (jnp.full((bsz, n), 1e10, jnp.float32),
         jnp.zeros((bsz, 1), jnp.int32)))


def _fps(xyz_c, npoint):
    bsz, _, n = xyz_c.shape
    return pl.pallas_call(
        functools.partial(_fps_body, npoint=npoint, n=n),
        out_shape=jax.ShapeDtypeStruct((bsz, npoint, 3), jnp.float32),
    )(xyz_c)


def _sa_body(nx_ref, xtt_ref, z_ref, w1x_ref, b1_ref, *refs,
             r2, k, ts, n, nlayers):
    wbs = refs[:2 * (nlayers - 1)]
    o_ref = refs[-1]

    nx = nx_ref[0]
    xtt = xtt_ref[0]
    xn2 = jnp.sum(xtt * xtt, axis=0, keepdims=True)
    d = (jnp.sum(nx * nx, axis=1, keepdims=True) + xn2
         - 2.0 * jnp.dot(nx, xtt, preferred_element_type=jnp.float32))
    ci = _cumsum_lanes((d <= r2).astype(jnp.int32), n)
    cm = jnp.where(d <= r2, ci, 0)

    rows = ts * k
    rio = jax.lax.broadcasted_iota(jnp.int32, (rows, 1), 0)
    ki = rio // ts + 1
    cmr = _tile_rows(cm, k)
    oh = (cmr == ki)
    empty = _tile_rows(ci[:, n - 1:n] == 0, k)
    lane = jax.lax.broadcasted_iota(jnp.int32, (rows, n), 1)
    oh = (oh | (empty & (lane == n - 1))).astype(jnp.float32)

    sub = _tile_rows(
        jnp.dot(nx, w1x_ref[...], preferred_element_type=jnp.float32),
        k)
    h = jnp.dot(oh, z_ref[0], preferred_element_type=jnp.float32,
                precision=jax.lax.Precision.HIGHEST)
    h = jnp.maximum(h - sub + b1_ref[...], 0.0)
    for j in range(nlayers - 1):
        w, b = wbs[2 * j][...], wbs[2 * j + 1][...]
        h = jnp.maximum(
            jnp.dot(h, w, preferred_element_type=jnp.float32) + b, 0.0)

    h = h + (jnp.sum(oh, axis=1, keepdims=True) - 1.0) * 1e30
    while rows > ts:
        rows //= 2
        h = jnp.maximum(h[:rows], h[rows:])
    o_ref[0] = h


def _sa_branch(nx, xtc, pt, radius, k, ts, wbs):
    bsz, s, _ = nx.shape
    n = xtc.shape[2]
    cout = wbs[-1][0].shape[1]
    nlayers = len(wbs)
    w1 = wbs[0][0]
    w1x = w1[-3:]
    xtr = jnp.transpose(xtc, (0, 2, 1))
    if pt is not None:
        z = (jnp.dot(pt, w1[:-3], precision=jax.lax.Precision.HIGHEST)
             + jnp.dot(xtr, w1x, precision=jax.lax.Precision.HIGHEST))
    else:
        z = jnp.dot(xtr, w1[:3] + w1x, precision=jax.lax.Precision.HIGHEST)
    in_specs = [
        pl.BlockSpec((1, ts, 3), lambda bi, ti: (bi, ti, 0)),
        pl.BlockSpec((1, 3, n), lambda bi, ti: (bi, 0, 0)),
        pl.BlockSpec((1, n, z.shape[2]), lambda bi, ti: (bi, 0, 0)),
        pl.BlockSpec(w1x.shape, lambda bi, ti: (0, 0)),
        pl.BlockSpec(wbs[0][1].shape, lambda bi, ti: (0, 0)),
    ]
    args = [nx, xtc, z, w1x, wbs[0][1]]
    for wt, b in wbs[1:]:
        in_specs.append(pl.BlockSpec(wt.shape, lambda bi, ti: (0, 0)))
        in_specs.append(pl.BlockSpec(b.shape, lambda bi, ti: (0, 0)))
        args += [wt, b]
    return pl.pallas_call(
        functools.partial(
            _sa_body, r2=float(np.float32(radius) ** 2), k=k, ts=ts, n=n,
            nlayers=nlayers),
        grid=(bsz, s // ts),
        in_specs=in_specs,
        out_specs=pl.BlockSpec((1, ts, cout), lambda bi, ti: (bi, ti, 0)),
        out_shape=jax.ShapeDtypeStruct((bsz, s, cout), jnp.float32),
    )(*args)


def _sa3fp3_body(x_ref, *refs):
    o_ref = refs[-1]
    x = x_ref[0]
    h = x
    for j in range(3):
        w = refs[2 * j][...]
        b = refs[2 * j + 1][...]
        h = jnp.maximum(
            jnp.dot(h, w, preferred_element_type=jnp.float32) + b, 0.0)
    g = jnp.max(h, axis=0, keepdims=True)
    f = jnp.concatenate(
        [x[:, 3:], jnp.broadcast_to(g, (x.shape[0], g.shape[1]))], axis=1)
    for j in range(3, 5):
        w = refs[2 * j][...]
        b = refs[2 * j + 1][...]
        f = jnp.maximum(
            jnp.dot(f, w, preferred_element_type=jnp.float32) + b, 0.0)
    o_ref[0] = f


def _sa3_fp3(l2cat, wbs):
    bsz, s, c = l2cat.shape
    cout = wbs[-1][0].shape[1]
    in_specs = [pl.BlockSpec((1, s, c), lambda bi: (bi, 0, 0))]
    args = [l2cat]
    for wt, b in wbs:
        in_specs.append(pl.BlockSpec(wt.shape, lambda bi: (0, 0)))
        in_specs.append(pl.BlockSpec(b.shape, lambda bi: (0, 0)))
        args += [wt, b]
    return pl.pallas_call(
        _sa3fp3_body,
        grid=(bsz,),
        in_specs=in_specs,
        out_specs=pl.BlockSpec((1, s, cout), lambda bi: (bi, 0, 0)),
        out_shape=jax.ShapeDtypeStruct((bsz, s, cout), jnp.float32),
    )(*args)


def _fp_body(x1_ref, p1_ref, x2_ref, p2_ref, *refs, nrelu, nlin, s):
    o_ref = refs[-1]
    x1 = x1_ref[0]
    x2 = x2_ref[0]
    tn = x1.shape[0]
    d = (jnp.sum(x1 * x1, axis=1, keepdims=True)
         + jnp.sum(x2 * x2, axis=1, keepdims=True).T
         - 2.0 * jnp.dot(x1, x2.T, preferred_element_type=jnp.float32))
    iota = jax.lax.broadcasted_iota(jnp.int32, (tn, s), 1)
    oh = jnp.zeros((tn, s), jnp.float32)
    recips = []
    onehots = []
    for _ in range(3):
        m = jnp.min(d, axis=1, keepdims=True)
        idx = jnp.min(jnp.where(d == m, iota, s), axis=1, keepdims=True)
        hit = (iota == idx)
        recips.append(1.0 / (m + 1e-8))
        onehots.append(hit)
        d = jnp.where(hit, jnp.inf, d)
    wsum = recips[0] + recips[1] + recips[2]
    for r, hit in zip(recips, onehots):
        oh = oh + jnp.where(hit, jnp.broadcast_to(r / wsum, hit.shape), 0.0)
    interp = jnp.dot(oh, p2_ref[0], preferred_element_type=jnp.float32,
                     precision=jax.lax.Precision.HIGHEST)
    h = jnp.concatenate([p1_ref[0], interp], axis=1)
    for j in range(nrelu + nlin):
        w = refs[2 * j][...]
        b = refs[2 * j + 1][...]
        h = jnp.dot(h, w, preferred_element_type=jnp.float32) + b
        if j < nrelu:
            h = jnp.maximum(h, 0.0)
    o_ref[0] = h


def _fp(x1, p1, x2, p2, wbs, nlin, tn):
    bsz, n, _ = x1.shape
    s = x2.shape[1]
    cout = wbs[-1][0].shape[1]
    in_specs = [
        pl.BlockSpec((1, tn, 3), lambda bi, ti: (bi, ti, 0)),
        pl.BlockSpec((1, tn, p1.shape[2]), lambda bi, ti: (bi, ti, 0)),
        pl.BlockSpec((1, s, 3), lambda bi, ti: (bi, 0, 0)),
        pl.BlockSpec((1, s, p2.shape[2]), lambda bi, ti: (bi, 0, 0)),
    ]
    args = [x1, p1, x2, p2]
    for wt, b in wbs:
        in_specs.append(pl.BlockSpec(wt.shape, lambda bi, ti: (0, 0)))
        in_specs.append(pl.BlockSpec(b.shape, lambda bi, ti: (0, 0)))
        args += [wt, b]
    return pl.pallas_call(
        functools.partial(_fp_body, nrelu=len(wbs) - nlin, nlin=nlin, s=s),
        grid=(bsz, n // tn),
        in_specs=in_specs,
        out_specs=pl.BlockSpec((1, tn, cout), lambda bi, ti: (bi, ti, 0)),
        out_shape=jax.ShapeDtypeStruct((bsz, n, cout), jnp.float32),
    )(*args)


def kernel(xyz, params):
    p = params
    xt = jnp.transpose(xyz, (0, 2, 1))

    nx1 = _fps(xyz, 512)
    outs = []
    for bi, (radius, k, ts) in enumerate(
            [(0.1, 32, 32), (0.2, 64, 16), (0.4, 128, 8)]):
        wbs = [_fold(p, 'sa1_%d_%d' % (bi, j)) for j in range(len(_SA1[bi]))]
        outs.append(_sa_branch(nx1, xyz, None, radius, k, ts, wbs))
    l1p = jnp.concatenate(outs, axis=-1)

    nx1c = jnp.transpose(nx1, (0, 2, 1))
    nx2 = _fps(nx1c, 128)
    outs = []
    for bi, (radius, k, ts) in enumerate([(0.4, 64, 16), (0.8, 128, 8)]):
        wbs = [_fold(p, 'sa2_%d_%d' % (bi, j)) for j in range(len(_SA2[bi]))]
        outs.append(_sa_branch(nx2, nx1c, l1p, radius, k, ts, wbs))
    l2p = jnp.concatenate(outs, axis=-1)

    l2cat = jnp.concatenate([nx2, l2p], axis=-1)
    wbs = ([_fold(p, 'sa3_%d' % j) for j in range(3)]
           + [_fold(p, 'fp3_%d' % j) for j in range(2)])
    l2p = _sa3_fp3(l2cat, wbs)

    wbs = [_fold(p, 'fp2_%d' % j) for j in range(2)]
    l1p = _fp(nx1, l1p, nx2, l2p, wbs, nlin=0, tn=512)

    wbs = ([_fold(p, 'fp1_%d' % j) for j in range(2)]
           + [_fold(p, 'conv1'), _fold(p, 'conv2')])
    return _fp(xt, xt, nx1, l1p, wbs, nlin=1, tn=512)
